# baseline (speedup 1.0000x reference)
import functools

import jax
import jax.numpy as jnp
from jax import lax
from jax.experimental import pallas as pl
from jax.experimental.pallas import tpu as pltpu
from jax.experimental.pallas import tpu_sc as plsc

NC = 2
NS = 16
NT = NC * NS
CHUNK = 6400


def _cdiv(a, b):
    return (a + b - 1) // b


def _rsqrt16(d):
    i = lax.bitcast_convert_type(d, jnp.int32)
    i = 0x5F3759DF - lax.shift_right_logical(i, 1)
    y = lax.bitcast_convert_type(i, jnp.float32)
    for _ in range(2):
        y = y * (1.5 - 0.5 * d * y * y)
    return y


def _make_sc_kernel(N, NP, SL, EPT, EPTA):
    mesh = plsc.VectorSubcoreMesh(
        core_axis_name="c", subcore_axis_name="s", num_cores=NC,
        num_subcores=NS)
    f32 = jnp.float32

    @functools.partial(
        pl.kernel,
        out_type=(jax.ShapeDtypeStruct((NC, NP), f32),
                  jax.ShapeDtypeStruct((NC, NP), f32),
                  jax.ShapeDtypeStruct((1, NP), f32)),
        mesh=mesh,
        scratch_types=[
            pltpu.VMEM((CHUNK,), jnp.int32),
            pltpu.VMEM((CHUNK,), jnp.int32),
            pltpu.VMEM((CHUNK,), jnp.int32),
            pltpu.VMEM((CHUNK,), jnp.int32),
            pltpu.VMEM((CHUNK,), f32),
            pltpu.VMEM((CHUNK,), f32),
            pltpu.VMEM((CHUNK,), f32),
            pltpu.VMEM((CHUNK,), f32),
            pltpu.VMEM((SL,), f32),
            pltpu.VMEM((SL,), f32),
            pltpu.VMEM_SHARED((NP,), f32),
            pltpu.VMEM_SHARED((NP,), f32),
            pltpu.VMEM_SHARED((NP,), f32),
            pltpu.VMEM_SHARED((NP,), f32),
        ] + [pltpu.SemaphoreType.DMA] * 10,
    )
    def sc_kernel(srcf_hbm, dstf_hbm, x_hbm,
                  sin_out, sout_out, dinv_out,
                  idst0_v, idst1_v, isrc0_v, isrc1_v,
                  uval0_v, uval1_v, dval0_v, dval1_v,
                  deg_v, x_v,
                  u_sh, dinv_sh, sin_sh, sout_sh,
                  sem_i0, sem_i1, sem_gu0, sem_gu1, sem_gd0, sem_gd1,
                  sem_s10, sem_s11, sem_s20, sem_s21):
        cid = lax.axis_index("c")
        sid = lax.axis_index("s")
        idst_v = (idst0_v, idst1_v)
        isrc_v = (isrc0_v, isrc1_v)
        uval_v = (uval0_v, uval1_v)
        dval_v = (dval0_v, dval1_v)
        sem_i = (sem_i0, sem_i1)
        sem_gu = (sem_gu0, sem_gu1)
        sem_gd = (sem_gd0, sem_gd1)
        sem_s1 = (sem_s10, sem_s11)
        sem_s2 = (sem_s20, sem_s21)

        lo = sid * SL
        sl = lambda: pl.ds(lo, SL)

        onev = jnp.ones((16,), f32)

        def ofill(i, carry):
            uval0_v[pl.ds(i * 16, 16)] = onev
            return carry

        lax.fori_loop(0, CHUNK // 16, ofill, 0)

        zvec = jnp.zeros((16,), f32)

        def zfill(i, carry):
            deg_v[pl.ds(i * 16, 16)] = zvec
            return carry

        lax.fori_loop(0, SL // 16, zfill, 0)
        pltpu.sync_copy(deg_v, sin_sh.at[sl()])
        pltpu.sync_copy(deg_v, sout_sh.at[sl()])
        plsc.subcore_barrier()

        nA = EPTA // CHUNK
        baseA = sid * EPTA
        idx_d = pltpu.async_copy(
            dstf_hbm.at[0, pl.ds(baseA, CHUNK)], idst_v[0], sem_i[0])
        scat = [None, None]
        for k in range(nA):
            p = k & 1
            q = p ^ 1
            if k + 1 < nA:
                if scat[q] is not None:
                    scat[q].wait()
                    scat[q] = None
                idx_next = pltpu.async_copy(
                    dstf_hbm.at[0, pl.ds(baseA + (k + 1) * CHUNK, CHUNK)],
                    idst_v[q], sem_i[q])
            idx_d.wait()
            scat[p] = pltpu.async_copy(
                uval0_v, sout_sh.at[idst_v[p]], sem_s1[p], add=True)
            if k + 1 < nA:
                idx_d = idx_next
        for d in scat:
            if d is not None:
                d.wait()
        plsc.subcore_barrier()

        pltpu.sync_copy(sout_sh.at[sl()], deg_v)
        pltpu.sync_copy(x_hbm.at[0, sl()], x_v)
        lanes = lax.iota(jnp.int32, 16)

        def a2_body(i, carry):
            off = pl.ds(i * 16, 16)
            d = deg_v[off] + 1.0
            y = _rsqrt16(d)
            gidx = lo + i * 16 + lanes
            y = jnp.where(gidx < N, y, 0.0)
            deg_v[off] = y
            x_v[off] = y * x_v[off]
            return carry

        lax.fori_loop(0, SL // 16, a2_body, 0)
        pltpu.sync_copy(deg_v, dinv_sh.at[sl()])
        pltpu.sync_copy(x_v, u_sh.at[sl()])

        @pl.when(cid == 0)
        def _():
            pltpu.sync_copy(deg_v, dinv_out.at[0, sl()])

        def zfill2(i, carry):
            x_v[pl.ds(i * 16, 16)] = zvec
            return carry

        lax.fori_loop(0, SL // 16, zfill2, 0)
        pltpu.sync_copy(x_v, sout_sh.at[sl()])
        plsc.subcore_barrier()

        nB = EPT // CHUNK
        baseB = (cid * NS + sid) * EPT
        i_s = pltpu.async_copy(
            srcf_hbm.at[0, pl.ds(baseB, CHUNK)], isrc_v[0], sem_i[0])
        i_d = pltpu.async_copy(
            dstf_hbm.at[0, pl.ds(baseB, CHUNK)], idst_v[0], sem_i[0])
        s1 = [None, None]
        s2 = [None, None]
        for k in range(nB):
            p = k & 1
            q = p ^ 1
            if k + 1 < nB:
                for s in (s1, s2):
                    if s[q] is not None:
                        s[q].wait()
                        s[q] = None
                off = baseB + (k + 1) * CHUNK
                i_sn = pltpu.async_copy(
                    srcf_hbm.at[0, pl.ds(off, CHUNK)], isrc_v[q], sem_i[q])
                i_dn = pltpu.async_copy(
                    dstf_hbm.at[0, pl.ds(off, CHUNK)], idst_v[q], sem_i[q])
            i_s.wait()
            i_d.wait()
            g_u = pltpu.async_copy(
                u_sh.at[isrc_v[p]], uval_v[p], sem_gu[p])
            g_d = pltpu.async_copy(
                dinv_sh.at[idst_v[p]], dval_v[p], sem_gd[p])
            g_u.wait()
            s1[p] = pltpu.async_copy(
                uval_v[p], sin_sh.at[idst_v[p]], sem_s1[p], add=True)
            g_d.wait()
            s2[p] = pltpu.async_copy(
                dval_v[p], sout_sh.at[isrc_v[p]], sem_s2[p], add=True)
            if k + 1 < nB:
                i_s = i_sn
                i_d = i_dn
        for s in (s1, s2):
            for d in s:
                if d is not None:
                    d.wait()
        plsc.subcore_barrier()

        pltpu.sync_copy(sin_sh.at[sl()], sin_out.at[cid, sl()])
        pltpu.sync_copy(sout_sh.at[sl()], sout_out.at[cid, sl()])

    return sc_kernel


def _split_body(nsp, N, NP, KSP, ei_ref, srcf_ref, dstf_ref):
    i = pl.program_id(0)

    @pl.when(i < nsp)
    def _():
        srcf_ref[...] = ei_ref[0:1, :]
        dstf_ref[...] = ei_ref[1:2, :]

    @pl.when(i >= nsp)
    def _():
        pad = N + lax.rem(lax.broadcasted_iota(jnp.int32, (1, KSP), 1),
                          NP - N)
        srcf_ref[...] = pad
        dstf_ref[...] = pad


def _dense_body(N, nsteps, sinp_ref, soutp_ref, dinv_ref, xp_ref, w1_ref,
                b1_ref, w2_ref, b2_ref, out_ref, vacc_ref):
    i = pl.program_id(0)
    dv = dinv_ref[...]
    s_in = sinp_ref[0:1, :] + sinp_ref[1:2, :]
    s_out = soutp_ref[0:1, :] + soutp_ref[1:2, :]
    agg = dv * s_in + dv * dv * xp_ref[...]
    cc = dv * s_out + dv * dv
    m = jnp.maximum(w1_ref[...] * agg + b1_ref[...], 0.0)
    w = jnp.sum(m * cc, axis=1, keepdims=True)

    @pl.when(i == 0)
    def _():
        vacc_ref[...] = jnp.zeros_like(vacc_ref)

    vacc_ref[...] += w

    @pl.when(i == nsteps - 1)
    def _():
        out_ref[...] = (
            lax.dot_general(vacc_ref[...] * (1.0 / N), w2_ref[...],
                            (((0,), (0,)), ((), ())),
                            preferred_element_type=jnp.float32)
            + b2_ref[...])


def kernel(x, edge_index, W1, b1, W2, b2):
    N = x.shape[0]
    E = edge_index.shape[1]
    hid = W1.shape[1]
    dout = W2.shape[1]

    NB = 7168
    NP = _cdiv(N, NB) * NB
    SL = NP // NS
    nsteps = NP // NB
    EP = _cdiv(E, NT * CHUNK) * NT * CHUNK
    EPT = EP // NT
    EPTA = EP // NS

    f32 = jnp.float32
    xf = jnp.concatenate([x[:, 0].astype(f32),
                          jnp.zeros((NP - N,), f32)]).reshape(1, NP)
    ei = edge_index.astype(jnp.int32)

    KSP = 12800
    assert E % KSP == 0 and EP % KSP == 0, (E, EP, KSP)
    nsp = E // KSP
    srcf, dstf = pl.pallas_call(
        functools.partial(_split_body, nsp, N, NP, KSP),
        grid=(EP // KSP,),
        in_specs=[pl.BlockSpec((2, KSP),
                               lambda i: (0, jnp.minimum(i, nsp - 1)))],
        out_specs=[pl.BlockSpec((1, KSP), lambda i: (0, i)),
                   pl.BlockSpec((1, KSP), lambda i: (0, i))],
        out_shape=(jax.ShapeDtypeStruct((1, EP), jnp.int32),
                   jax.ShapeDtypeStruct((1, EP), jnp.int32)),
    )(ei)

    sinp, soutp, dinv = _make_sc_kernel(N, NP, SL, EPT, EPTA)(
        srcf, dstf, xf)

    out2d = pl.pallas_call(
        functools.partial(_dense_body, N, nsteps),
        grid=(nsteps,),
        in_specs=[
            pl.BlockSpec((NC, NB), lambda i: (0, i)),
            pl.BlockSpec((NC, NB), lambda i: (0, i)),
            pl.BlockSpec((1, NB), lambda i: (0, i)),
            pl.BlockSpec((1, NB), lambda i: (0, i)),
            pl.BlockSpec((hid, 1), lambda i: (0, 0)),
            pl.BlockSpec((hid, 1), lambda i: (0, 0)),
            pl.BlockSpec((hid, dout), lambda i: (0, 0)),
            pl.BlockSpec((1, dout), lambda i: (0, 0)),
        ],
        out_specs=pl.BlockSpec((1, dout), lambda i: (0, 0)),
        out_shape=jax.ShapeDtypeStruct((1, dout), f32),
        scratch_shapes=[pltpu.VMEM((hid, 1), f32)],
    )(sinp, soutp, dinv, xf,
      W1.reshape(hid, 1).astype(f32), b1.reshape(hid, 1).astype(f32),
      W2.astype(f32), b2.reshape(1, dout).astype(f32))

    return out2d.reshape(dout)


# speedup vs baseline: 1.4593x; 1.4593x over previous
import functools

import jax
import jax.numpy as jnp
from jax import lax
from jax.experimental import pallas as pl
from jax.experimental.pallas import tpu as pltpu
from jax.experimental.pallas import tpu_sc as plsc

NC = 2
NS = 16
NT = NC * NS
CHUNK = 10000


def _cdiv(a, b):
    return (a + b - 1) // b


def _rsqrt16(d):
    i = lax.bitcast_convert_type(d, jnp.int32)
    i = 0x5F3759DF - lax.shift_right_logical(i, 1)
    y = lax.bitcast_convert_type(i, jnp.float32)
    for _ in range(2):
        y = y * (1.5 - 0.5 * d * y * y)
    return y


def _make_sc_kernel(N, NP, SL, EPT, EPTA):
    mesh = plsc.VectorSubcoreMesh(
        core_axis_name="c", subcore_axis_name="s", num_cores=NC,
        num_subcores=NS)
    f32 = jnp.float32

    @functools.partial(
        pl.kernel,
        out_type=(jax.ShapeDtypeStruct((NC, NP), f32),
                  jax.ShapeDtypeStruct((NC, NP), f32),
                  jax.ShapeDtypeStruct((1, NP), f32)),
        mesh=mesh,
        scratch_types=[
            pltpu.VMEM((CHUNK,), jnp.int32),
            pltpu.VMEM((CHUNK,), jnp.int32),
            pltpu.VMEM((CHUNK,), jnp.int32),
            pltpu.VMEM((CHUNK,), jnp.int32),
            pltpu.VMEM((CHUNK,), f32),
            pltpu.VMEM((CHUNK,), f32),
            pltpu.VMEM((CHUNK,), f32),
            pltpu.VMEM((CHUNK,), f32),
            pltpu.VMEM((SL,), f32),
            pltpu.VMEM((SL,), f32),
            pltpu.VMEM_SHARED((NP,), f32),
            pltpu.VMEM_SHARED((NP,), f32),
            pltpu.VMEM_SHARED((NP,), f32),
            pltpu.VMEM_SHARED((NP,), f32),
        ] + [pltpu.SemaphoreType.DMA] * 10,
    )
    def sc_kernel(srcf_hbm, dstf_hbm, x_hbm,
                  sin_out, sout_out, dinv_out,
                  idst0_v, idst1_v, isrc0_v, isrc1_v,
                  uval0_v, uval1_v, dval0_v, dval1_v,
                  deg_v, x_v,
                  u_sh, dinv_sh, sin_sh, sout_sh,
                  sem_i0, sem_i1, sem_gu0, sem_gu1, sem_gd0, sem_gd1,
                  sem_s10, sem_s11, sem_s20, sem_s21):
        cid = lax.axis_index("c")
        sid = lax.axis_index("s")
        idst_v = (idst0_v, idst1_v)
        isrc_v = (isrc0_v, isrc1_v)
        uval_v = (uval0_v, uval1_v)
        dval_v = (dval0_v, dval1_v)
        sem_i = (sem_i0, sem_i1)
        sem_gu = (sem_gu0, sem_gu1)
        sem_gd = (sem_gd0, sem_gd1)
        sem_s1 = (sem_s10, sem_s11)
        sem_s2 = (sem_s20, sem_s21)

        lo = sid * SL
        sl = lambda: pl.ds(lo, SL)

        onev = jnp.ones((16,), f32)

        def ofill(i, carry):
            uval0_v[pl.ds(i * 16, 16)] = onev
            return carry

        lax.fori_loop(0, CHUNK // 16, ofill, 0)

        zvec = jnp.zeros((16,), f32)

        def zfill(i, carry):
            deg_v[pl.ds(i * 16, 16)] = zvec
            return carry

        lax.fori_loop(0, SL // 16, zfill, 0)
        pltpu.sync_copy(deg_v, sin_sh.at[sl()])
        pltpu.sync_copy(deg_v, sout_sh.at[sl()])
        plsc.subcore_barrier()

        nA = EPTA // CHUNK
        baseA = sid * EPTA
        idx_d = pltpu.async_copy(
            dstf_hbm.at[pl.ds(baseA, CHUNK)], idst_v[0], sem_i[0])
        scat = [None, None]
        for k in range(nA):
            p = k & 1
            q = p ^ 1
            if k + 1 < nA:
                if scat[q] is not None:
                    scat[q].wait()
                    scat[q] = None
                idx_next = pltpu.async_copy(
                    dstf_hbm.at[pl.ds(baseA + (k + 1) * CHUNK, CHUNK)],
                    idst_v[q], sem_i[q])
            idx_d.wait()
            scat[p] = pltpu.async_copy(
                uval0_v, sout_sh.at[idst_v[p]], sem_s1[p], add=True)
            if k + 1 < nA:
                idx_d = idx_next
        for d in scat:
            if d is not None:
                d.wait()
        plsc.subcore_barrier()

        pltpu.sync_copy(sout_sh.at[sl()], deg_v)
        pltpu.sync_copy(x_hbm.at[0, sl()], x_v)
        lanes = lax.iota(jnp.int32, 16)

        def a2_body(i, carry):
            off = pl.ds(i * 16, 16)
            d = deg_v[off] + 1.0
            y = _rsqrt16(d)
            gidx = lo + i * 16 + lanes
            y = jnp.where(gidx < N, y, 0.0)
            deg_v[off] = y
            x_v[off] = y * x_v[off]
            return carry

        lax.fori_loop(0, SL // 16, a2_body, 0)
        pltpu.sync_copy(deg_v, dinv_sh.at[sl()])
        pltpu.sync_copy(x_v, u_sh.at[sl()])

        @pl.when(cid == 0)
        def _():
            pltpu.sync_copy(deg_v, dinv_out.at[0, sl()])

        def zfill2(i, carry):
            x_v[pl.ds(i * 16, 16)] = zvec
            return carry

        lax.fori_loop(0, SL // 16, zfill2, 0)
        pltpu.sync_copy(x_v, sout_sh.at[sl()])
        plsc.subcore_barrier()

        nB = EPT // CHUNK
        baseB = (cid * NS + sid) * EPT
        i_s = pltpu.async_copy(
            srcf_hbm.at[pl.ds(baseB, CHUNK)], isrc_v[0], sem_i[0])
        i_d = pltpu.async_copy(
            dstf_hbm.at[pl.ds(baseB, CHUNK)], idst_v[0], sem_i[0])
        s1 = [None, None]
        s2 = [None, None]
        for k in range(nB):
            p = k & 1
            q = p ^ 1
            if k + 1 < nB:
                for s in (s1, s2):
                    if s[q] is not None:
                        s[q].wait()
                        s[q] = None
                off = baseB + (k + 1) * CHUNK
                i_sn = pltpu.async_copy(
                    srcf_hbm.at[pl.ds(off, CHUNK)], isrc_v[q], sem_i[q])
                i_dn = pltpu.async_copy(
                    dstf_hbm.at[pl.ds(off, CHUNK)], idst_v[q], sem_i[q])
            i_s.wait()
            i_d.wait()
            g_u = pltpu.async_copy(
                u_sh.at[isrc_v[p]], uval_v[p], sem_gu[p])
            g_d = pltpu.async_copy(
                dinv_sh.at[idst_v[p]], dval_v[p], sem_gd[p])
            g_u.wait()
            s1[p] = pltpu.async_copy(
                uval_v[p], sin_sh.at[idst_v[p]], sem_s1[p], add=True)
            g_d.wait()
            s2[p] = pltpu.async_copy(
                dval_v[p], sout_sh.at[isrc_v[p]], sem_s2[p], add=True)
            if k + 1 < nB:
                i_s = i_sn
                i_d = i_dn
        for s in (s1, s2):
            for d in s:
                if d is not None:
                    d.wait()
        plsc.subcore_barrier()

        pltpu.sync_copy(sin_sh.at[sl()], sin_out.at[cid, sl()])
        pltpu.sync_copy(sout_sh.at[sl()], sout_out.at[cid, sl()])

    return sc_kernel


def _split_body(ei_ref, srcf_ref, dstf_ref):
    srcf_ref[...] = ei_ref[0]
    dstf_ref[...] = ei_ref[1]


def _dense_body(N, nsteps, sinp_ref, soutp_ref, dinv_ref, xp_ref, w1_ref,
                b1_ref, w2_ref, b2_ref, out_ref, vacc_ref):
    i = pl.program_id(0)
    dv = dinv_ref[...]
    s_in = sinp_ref[0:1, :] + sinp_ref[1:2, :]
    s_out = soutp_ref[0:1, :] + soutp_ref[1:2, :]
    agg = dv * s_in + dv * dv * xp_ref[...]
    cc = dv * s_out + dv * dv
    m = jnp.maximum(w1_ref[...] * agg + b1_ref[...], 0.0)
    w = jnp.sum(m * cc, axis=1, keepdims=True)

    @pl.when(i == 0)
    def _():
        vacc_ref[...] = jnp.zeros_like(vacc_ref)

    vacc_ref[...] += w

    @pl.when(i == nsteps - 1)
    def _():
        out_ref[...] = (
            lax.dot_general(vacc_ref[...] * (1.0 / N), w2_ref[...],
                            (((0,), (0,)), ((), ())),
                            preferred_element_type=jnp.float32)
            + b2_ref[...])


def kernel(x, edge_index, W1, b1, W2, b2):
    N = x.shape[0]
    E = edge_index.shape[1]
    hid = W1.shape[1]
    dout = W2.shape[1]

    NB = 7168
    NP = _cdiv(N, NB) * NB
    SL = NP // NS
    nsteps = NP // NB
    EP = _cdiv(E, NT * CHUNK) * NT * CHUNK
    EPT = EP // NT
    EPTA = EP // NS

    f32 = jnp.float32
    xf = jnp.concatenate([x[:, 0].astype(f32),
                          jnp.zeros((NP - N,), f32)]).reshape(1, NP)
    ei = edge_index.astype(jnp.int32)

    assert EP == E, (E, EP)
    srcf, dstf = pl.pallas_call(
        _split_body,
        out_shape=(jax.ShapeDtypeStruct((E,), jnp.int32),
                   jax.ShapeDtypeStruct((E,), jnp.int32)),
    )(ei)

    sinp, soutp, dinv = _make_sc_kernel(N, NP, SL, EPT, EPTA)(
        srcf, dstf, xf)

    out2d = pl.pallas_call(
        functools.partial(_dense_body, N, nsteps),
        grid=(nsteps,),
        in_specs=[
            pl.BlockSpec((NC, NB), lambda i: (0, i)),
            pl.BlockSpec((NC, NB), lambda i: (0, i)),
            pl.BlockSpec((1, NB), lambda i: (0, i)),
            pl.BlockSpec((1, NB), lambda i: (0, i)),
            pl.BlockSpec((hid, 1), lambda i: (0, 0)),
            pl.BlockSpec((hid, 1), lambda i: (0, 0)),
            pl.BlockSpec((hid, dout), lambda i: (0, 0)),
            pl.BlockSpec((1, dout), lambda i: (0, 0)),
        ],
        out_specs=pl.BlockSpec((1, dout), lambda i: (0, 0)),
        out_shape=jax.ShapeDtypeStruct((1, dout), f32),
        scratch_shapes=[pltpu.VMEM((hid, 1), f32)],
    )(sinp, soutp, dinv, xf,
      W1.reshape(hid, 1).astype(f32), b1.reshape(hid, 1).astype(f32),
      W2.astype(f32), b2.reshape(1, dout).astype(f32))

    return out2d.reshape(dout)


# speedup vs baseline: 1.6742x; 1.1473x over previous
import functools

import jax
import jax.numpy as jnp
from jax import lax
from jax.experimental import pallas as pl
from jax.experimental.pallas import tpu as pltpu
from jax.experimental.pallas import tpu_sc as plsc

NC = 2
NS = 16
NT = NC * NS
CHUNK = 10000


def _cdiv(a, b):
    return (a + b - 1) // b


def _split_body(ei_ref, srcf_ref, dstf_ref):
    srcf_ref[...] = ei_ref[0]
    dstf_ref[...] = ei_ref[1]


def _make_hist_kernel(NP, SL, EPT):
    mesh = plsc.VectorSubcoreMesh(
        core_axis_name="c", subcore_axis_name="s", num_cores=NC,
        num_subcores=NS)
    f32 = jnp.float32

    @functools.partial(
        pl.kernel,
        out_type=jax.ShapeDtypeStruct((NC, NP), f32),
        mesh=mesh,
        scratch_types=[
            pltpu.VMEM((CHUNK,), jnp.int32),
            pltpu.VMEM((CHUNK,), jnp.int32),
            pltpu.VMEM((CHUNK,), f32),
            pltpu.VMEM((SL,), f32),
            pltpu.VMEM_SHARED((NP,), f32),
        ] + [pltpu.SemaphoreType.DMA] * 4,
    )
    def hist_kernel(dstf_hbm, degp_out,
                    idst0_v, idst1_v, ones_v, z_v, deg_sh,
                    sem_i0, sem_i1, sem_s0, sem_s1):
        cid = lax.axis_index("c")
        sid = lax.axis_index("s")
        idst_v = (idst0_v, idst1_v)
        sem_i = (sem_i0, sem_i1)
        sem_s = (sem_s0, sem_s1)

        lo = sid * SL
        sl = lambda: pl.ds(lo, SL)
        onev = jnp.ones((16,), f32)
        zvec = jnp.zeros((16,), f32)

        def fill(i, carry):
            ones_v[pl.ds(i * 16, 16)] = onev
            return carry

        lax.fori_loop(0, CHUNK // 16, fill, 0)

        def zfill(i, carry):
            z_v[pl.ds(i * 16, 16)] = zvec
            return carry

        lax.fori_loop(0, SL // 16, zfill, 0)
        pltpu.sync_copy(z_v, deg_sh.at[sl()])
        plsc.subcore_barrier()

        nA = EPT // CHUNK
        base = (cid * NS + sid) * EPT
        idx_d = pltpu.async_copy(
            dstf_hbm.at[pl.ds(base, CHUNK)], idst_v[0], sem_i[0])
        scat = [None, None]
        for k in range(nA):
            p = k & 1
            q = p ^ 1
            if k + 1 < nA:
                if scat[q] is not None:
                    scat[q].wait()
                    scat[q] = None
                idx_next = pltpu.async_copy(
                    dstf_hbm.at[pl.ds(base + (k + 1) * CHUNK, CHUNK)],
                    idst_v[q], sem_i[q])
            idx_d.wait()
            scat[p] = pltpu.async_copy(
                ones_v, deg_sh.at[idst_v[p]], sem_s[p], add=True)
            if k + 1 < nA:
                idx_d = idx_next
        for d in scat:
            if d is not None:
                d.wait()
        plsc.subcore_barrier()
        pltpu.sync_copy(deg_sh.at[sl()], degp_out.at[cid, sl()])

    return hist_kernel


def _norm_body(N, degp_ref, xp_ref, u_ref, dinv_ref):
    deg = degp_ref[0:1, :] + degp_ref[1:2, :] + 1.0
    lin = lax.broadcasted_iota(jnp.int32, deg.shape, 1)
    dv = jnp.where(lin < N, lax.rsqrt(deg), 0.0)
    dinv_ref[...] = dv
    u_ref[...] = dv * xp_ref[...]


def _make_edge_kernel(NP, SL, EPT):
    mesh = plsc.VectorSubcoreMesh(
        core_axis_name="c", subcore_axis_name="s", num_cores=NC,
        num_subcores=NS)
    f32 = jnp.float32

    @functools.partial(
        pl.kernel,
        out_type=(jax.ShapeDtypeStruct((NC, NP), f32),
                  jax.ShapeDtypeStruct((NC, NP), f32)),
        mesh=mesh,
        scratch_types=[
            pltpu.VMEM((CHUNK,), jnp.int32),
            pltpu.VMEM((CHUNK,), jnp.int32),
            pltpu.VMEM((CHUNK,), jnp.int32),
            pltpu.VMEM((CHUNK,), jnp.int32),
            pltpu.VMEM((CHUNK,), f32),
            pltpu.VMEM((CHUNK,), f32),
            pltpu.VMEM((CHUNK,), f32),
            pltpu.VMEM((CHUNK,), f32),
            pltpu.VMEM((SL,), f32),
            pltpu.VMEM_SHARED((NP,), f32),
            pltpu.VMEM_SHARED((NP,), f32),
            pltpu.VMEM_SHARED((NP,), f32),
            pltpu.VMEM_SHARED((NP,), f32),
        ] + [pltpu.SemaphoreType.DMA] * 10,
    )
    def edge_kernel(srcf_hbm, dstf_hbm, u_hbm, dinv_hbm,
                    sin_out, sout_out,
                    idst0_v, idst1_v, isrc0_v, isrc1_v,
                    uval0_v, uval1_v, dval0_v, dval1_v, z_v,
                    u_sh, dinv_sh, sin_sh, sout_sh,
                    sem_i0, sem_i1, sem_gu0, sem_gu1, sem_gd0, sem_gd1,
                    sem_s10, sem_s11, sem_s20, sem_s21):
        cid = lax.axis_index("c")
        sid = lax.axis_index("s")
        idst_v = (idst0_v, idst1_v)
        isrc_v = (isrc0_v, isrc1_v)
        uval_v = (uval0_v, uval1_v)
        dval_v = (dval0_v, dval1_v)
        sem_i = (sem_i0, sem_i1)
        sem_gu = (sem_gu0, sem_gu1)
        sem_gd = (sem_gd0, sem_gd1)
        sem_s1 = (sem_s10, sem_s11)
        sem_s2 = (sem_s20, sem_s21)

        lo = sid * SL
        sl = lambda: pl.ds(lo, SL)

        tab_u = pltpu.async_copy(u_hbm.at[0, sl()], u_sh.at[sl()],
                                 sem_gu0)
        tab_d = pltpu.async_copy(dinv_hbm.at[0, sl()], dinv_sh.at[sl()],
                                 sem_gd0)
        zvec = jnp.zeros((16,), f32)

        def zfill(i, carry):
            z_v[pl.ds(i * 16, 16)] = zvec
            return carry

        lax.fori_loop(0, SL // 16, zfill, 0)
        pltpu.sync_copy(z_v, sin_sh.at[sl()])
        pltpu.sync_copy(z_v, sout_sh.at[sl()])
        tab_u.wait()
        tab_d.wait()
        plsc.subcore_barrier()

        nB = EPT // CHUNK
        base = (cid * NS + sid) * EPT
        i_s = pltpu.async_copy(
            srcf_hbm.at[pl.ds(base, CHUNK)], isrc_v[0], sem_i[0])
        i_d = pltpu.async_copy(
            dstf_hbm.at[pl.ds(base, CHUNK)], idst_v[0], sem_i[0])
        s1 = [None, None]
        s2 = [None, None]
        for k in range(nB):
            p = k & 1
            q = p ^ 1
            if k + 1 < nB:
                for s in (s1, s2):
                    if s[q] is not None:
                        s[q].wait()
                        s[q] = None
                off = base + (k + 1) * CHUNK
                i_sn = pltpu.async_copy(
                    srcf_hbm.at[pl.ds(off, CHUNK)], isrc_v[q], sem_i[q])
                i_dn = pltpu.async_copy(
                    dstf_hbm.at[pl.ds(off, CHUNK)], idst_v[q], sem_i[q])
            i_s.wait()
            i_d.wait()
            g_u = pltpu.async_copy(
                u_sh.at[isrc_v[p]], uval_v[p], sem_gu[p])
            g_d = pltpu.async_copy(
                dinv_sh.at[idst_v[p]], dval_v[p], sem_gd[p])
            g_u.wait()
            s1[p] = pltpu.async_copy(
                uval_v[p], sin_sh.at[idst_v[p]], sem_s1[p], add=True)
            g_d.wait()
            s2[p] = pltpu.async_copy(
                dval_v[p], sout_sh.at[isrc_v[p]], sem_s2[p], add=True)
            if k + 1 < nB:
                i_s = i_sn
                i_d = i_dn
        for s in (s1, s2):
            for d in s:
                if d is not None:
                    d.wait()
        plsc.subcore_barrier()

        pltpu.sync_copy(sin_sh.at[sl()], sin_out.at[cid, sl()])
        pltpu.sync_copy(sout_sh.at[sl()], sout_out.at[cid, sl()])

    return edge_kernel


def _dense_body(N, nsteps, sinp_ref, soutp_ref, dinv_ref, xp_ref, w1_ref,
                b1_ref, w2_ref, b2_ref, out_ref, vacc_ref):
    i = pl.program_id(0)
    dv = dinv_ref[...]
    s_in = sinp_ref[0:1, :] + sinp_ref[1:2, :]
    s_out = soutp_ref[0:1, :] + soutp_ref[1:2, :]
    agg = dv * s_in + dv * dv * xp_ref[...]
    cc = dv * s_out + dv * dv
    m = jnp.maximum(w1_ref[...] * agg + b1_ref[...], 0.0)
    w = jnp.sum(m * cc, axis=1, keepdims=True)

    @pl.when(i == 0)
    def _():
        vacc_ref[...] = jnp.zeros_like(vacc_ref)

    vacc_ref[...] += w

    @pl.when(i == nsteps - 1)
    def _():
        out_ref[...] = (
            lax.dot_general(vacc_ref[...] * (1.0 / N), w2_ref[...],
                            (((0,), (0,)), ((), ())),
                            preferred_element_type=jnp.float32)
            + b2_ref[...])


def kernel(x, edge_index, W1, b1, W2, b2):
    N = x.shape[0]
    E = edge_index.shape[1]
    hid = W1.shape[1]
    dout = W2.shape[1]

    NB = 7168
    NP = _cdiv(N, NB) * NB
    SL = NP // NS
    nsteps = NP // NB
    assert E % (NT * CHUNK) == 0, (E, NT * CHUNK)
    EPT = E // NT

    f32 = jnp.float32
    xf = jnp.concatenate([x[:, 0].astype(f32),
                          jnp.zeros((NP - N,), f32)]).reshape(1, NP)
    ei = edge_index.astype(jnp.int32)

    srcf, dstf = pl.pallas_call(
        _split_body,
        out_shape=(jax.ShapeDtypeStruct((E,), jnp.int32),
                   jax.ShapeDtypeStruct((E,), jnp.int32)),
    )(ei)

    degp = _make_hist_kernel(NP, SL, EPT)(dstf)

    u, dinv = pl.pallas_call(
        functools.partial(_norm_body, N),
        out_shape=(jax.ShapeDtypeStruct((1, NP), f32),
                   jax.ShapeDtypeStruct((1, NP), f32)),
    )(degp, xf)

    sinp, soutp = _make_edge_kernel(NP, SL, EPT)(srcf, dstf, u, dinv)

    out2d = pl.pallas_call(
        functools.partial(_dense_body, N, nsteps),
        grid=(nsteps,),
        in_specs=[
            pl.BlockSpec((NC, NB), lambda i: (0, i)),
            pl.BlockSpec((NC, NB), lambda i: (0, i)),
            pl.BlockSpec((1, NB), lambda i: (0, i)),
            pl.BlockSpec((1, NB), lambda i: (0, i)),
            pl.BlockSpec((hid, 1), lambda i: (0, 0)),
            pl.BlockSpec((hid, 1), lambda i: (0, 0)),
            pl.BlockSpec((hid, dout), lambda i: (0, 0)),
            pl.BlockSpec((1, dout), lambda i: (0, 0)),
        ],
        out_specs=pl.BlockSpec((1, dout), lambda i: (0, 0)),
        out_shape=jax.ShapeDtypeStruct((1, dout), f32),
        scratch_shapes=[pltpu.VMEM((hid, 1), f32)],
    )(sinp, soutp, dinv, xf,
      W1.reshape(hid, 1).astype(f32), b1.reshape(hid, 1).astype(f32),
      W2.astype(f32), b2.reshape(1, dout).astype(f32))

    return out2d.reshape(dout)


# speedup vs baseline: 1.6902x; 1.0095x over previous
import functools

import jax
import jax.numpy as jnp
from jax import lax
from jax.experimental import pallas as pl
from jax.experimental.pallas import tpu as pltpu
from jax.experimental.pallas import tpu_sc as plsc

NC = 2
NS = 16
NT = NC * NS
CHUNK = 10000


def _cdiv(a, b):
    return (a + b - 1) // b


def _split_body(ei_ref, srcf_ref, dstf_ref):
    srcf_ref[...] = ei_ref[0]
    dstf_ref[...] = ei_ref[1]


def _make_hist_kernel(NP, SL, EPT):
    mesh = plsc.VectorSubcoreMesh(
        core_axis_name="c", subcore_axis_name="s", num_cores=NC,
        num_subcores=NS)
    f32 = jnp.float32

    @functools.partial(
        pl.kernel,
        out_type=jax.ShapeDtypeStruct((NC, NP), f32),
        mesh=mesh,
        scratch_types=[
            pltpu.VMEM((CHUNK,), jnp.int32),
            pltpu.VMEM((CHUNK,), jnp.int32),
            pltpu.VMEM((CHUNK,), f32),
            pltpu.VMEM((SL,), f32),
            pltpu.VMEM_SHARED((NP,), f32),
        ] + [pltpu.SemaphoreType.DMA] * 4,
    )
    def hist_kernel(dstf_hbm, degp_out,
                    idst0_v, idst1_v, ones_v, z_v, deg_sh,
                    sem_i0, sem_i1, sem_s0, sem_s1):
        cid = lax.axis_index("c")
        sid = lax.axis_index("s")
        idst_v = (idst0_v, idst1_v)
        sem_i = (sem_i0, sem_i1)
        sem_s = (sem_s0, sem_s1)

        lo = sid * SL
        sl = lambda: pl.ds(lo, SL)
        nA = EPT // CHUNK
        base = (cid * NS + sid) * EPT
        idx_d = pltpu.async_copy(
            dstf_hbm.at[pl.ds(base, CHUNK)], idst_v[0], sem_i[0])
        onev = jnp.ones((16,), f32)
        zvec = jnp.zeros((16,), f32)

        def fill(i, carry):
            ones_v[pl.ds(i * 16, 16)] = onev
            return carry

        lax.fori_loop(0, CHUNK // 16, fill, 0)

        def zfill(i, carry):
            z_v[pl.ds(i * 16, 16)] = zvec
            return carry

        lax.fori_loop(0, SL // 16, zfill, 0)
        pltpu.sync_copy(z_v, deg_sh.at[sl()])
        plsc.subcore_barrier()

        scat = [None, None]
        for k in range(nA):
            p = k & 1
            q = p ^ 1
            if k + 1 < nA:
                if scat[q] is not None:
                    scat[q].wait()
                    scat[q] = None
                idx_next = pltpu.async_copy(
                    dstf_hbm.at[pl.ds(base + (k + 1) * CHUNK, CHUNK)],
                    idst_v[q], sem_i[q])
            idx_d.wait()
            scat[p] = pltpu.async_copy(
                ones_v, deg_sh.at[idst_v[p]], sem_s[p], add=True)
            if k + 1 < nA:
                idx_d = idx_next
        for d in scat:
            if d is not None:
                d.wait()
        plsc.subcore_barrier()
        pltpu.sync_copy(deg_sh.at[sl()], degp_out.at[cid, sl()])

    return hist_kernel


def _norm_body(N, degp_ref, xp_ref, u_ref, dinv_ref, dinv2_ref):
    deg = degp_ref[0:1, :] + degp_ref[1:2, :] + 1.0
    lin = lax.broadcasted_iota(jnp.int32, deg.shape, 1)
    dv = jnp.where(lin < N, lax.rsqrt(deg), 0.0)
    dinv_ref[...] = dv[0]
    dinv2_ref[...] = dv
    u_ref[...] = (dv * xp_ref[...])[0]


def _make_edge_kernel(NP, SL, EPT):
    mesh = plsc.VectorSubcoreMesh(
        core_axis_name="c", subcore_axis_name="s", num_cores=NC,
        num_subcores=NS)
    f32 = jnp.float32

    @functools.partial(
        pl.kernel,
        out_type=(jax.ShapeDtypeStruct((NC, NP), f32),
                  jax.ShapeDtypeStruct((NC, NP), f32)),
        mesh=mesh,
        scratch_types=[
            pltpu.VMEM((CHUNK,), jnp.int32),
            pltpu.VMEM((CHUNK,), jnp.int32),
            pltpu.VMEM((CHUNK,), jnp.int32),
            pltpu.VMEM((CHUNK,), jnp.int32),
            pltpu.VMEM((CHUNK,), f32),
            pltpu.VMEM((CHUNK,), f32),
            pltpu.VMEM((CHUNK,), f32),
            pltpu.VMEM((CHUNK,), f32),
            pltpu.VMEM((SL,), f32),
            pltpu.VMEM_SHARED((NP,), f32),
            pltpu.VMEM_SHARED((NP,), f32),
            pltpu.VMEM_SHARED((NP,), f32),
            pltpu.VMEM_SHARED((NP,), f32),
        ] + [pltpu.SemaphoreType.DMA] * 10,
    )
    def edge_kernel(srcf_hbm, dstf_hbm, u_hbm, dinv_hbm,
                    sin_out, sout_out,
                    idst0_v, idst1_v, isrc0_v, isrc1_v,
                    uval0_v, uval1_v, dval0_v, dval1_v, z_v,
                    u_sh, dinv_sh, sin_sh, sout_sh,
                    sem_i0, sem_i1, sem_gu0, sem_gu1, sem_gd0, sem_gd1,
                    sem_s10, sem_s11, sem_s20, sem_s21):
        cid = lax.axis_index("c")
        sid = lax.axis_index("s")
        idst_v = (idst0_v, idst1_v)
        isrc_v = (isrc0_v, isrc1_v)
        uval_v = (uval0_v, uval1_v)
        dval_v = (dval0_v, dval1_v)
        sem_i = (sem_i0, sem_i1)
        sem_gu = (sem_gu0, sem_gu1)
        sem_gd = (sem_gd0, sem_gd1)
        sem_s1 = (sem_s10, sem_s11)
        sem_s2 = (sem_s20, sem_s21)

        lo = sid * SL
        sl = lambda: pl.ds(lo, SL)

        tab_u = pltpu.async_copy(u_hbm.at[sl()], u_sh.at[sl()],
                                 sem_gu0)
        tab_d = pltpu.async_copy(dinv_hbm.at[sl()], dinv_sh.at[sl()],
                                 sem_gd0)
        zvec = jnp.zeros((16,), f32)

        def zfill(i, carry):
            z_v[pl.ds(i * 16, 16)] = zvec
            return carry

        lax.fori_loop(0, SL // 16, zfill, 0)
        pltpu.sync_copy(z_v, sin_sh.at[sl()])
        pltpu.sync_copy(z_v, sout_sh.at[sl()])
        tab_u.wait()
        tab_d.wait()
        plsc.subcore_barrier()

        nB = EPT // CHUNK
        base = (cid * NS + sid) * EPT
        i_s = pltpu.async_copy(
            srcf_hbm.at[pl.ds(base, CHUNK)], isrc_v[0], sem_i[0])
        i_d = pltpu.async_copy(
            dstf_hbm.at[pl.ds(base, CHUNK)], idst_v[0], sem_i[0])
        s1 = [None, None]
        s2 = [None, None]
        for k in range(nB):
            p = k & 1
            q = p ^ 1
            if k + 1 < nB:
                for s in (s1, s2):
                    if s[q] is not None:
                        s[q].wait()
                        s[q] = None
                off = base + (k + 1) * CHUNK
                i_sn = pltpu.async_copy(
                    srcf_hbm.at[pl.ds(off, CHUNK)], isrc_v[q], sem_i[q])
                i_dn = pltpu.async_copy(
                    dstf_hbm.at[pl.ds(off, CHUNK)], idst_v[q], sem_i[q])
            i_s.wait()
            i_d.wait()
            g_u = pltpu.async_copy(
                u_sh.at[isrc_v[p]], uval_v[p], sem_gu[p])
            g_d = pltpu.async_copy(
                dinv_sh.at[idst_v[p]], dval_v[p], sem_gd[p])
            g_u.wait()
            s1[p] = pltpu.async_copy(
                uval_v[p], sin_sh.at[idst_v[p]], sem_s1[p], add=True)
            g_d.wait()
            s2[p] = pltpu.async_copy(
                dval_v[p], sout_sh.at[isrc_v[p]], sem_s2[p], add=True)
            if k + 1 < nB:
                i_s = i_sn
                i_d = i_dn
        for s in (s1, s2):
            for d in s:
                if d is not None:
                    d.wait()
        plsc.subcore_barrier()

        pltpu.sync_copy(sin_sh.at[sl()], sin_out.at[cid, sl()])
        pltpu.sync_copy(sout_sh.at[sl()], sout_out.at[cid, sl()])

    return edge_kernel


def _dense_body(N, nsteps, sinp_ref, soutp_ref, dinv_ref, xp_ref, w1_ref,
                b1_ref, w2_ref, b2_ref, out_ref, vacc_ref):
    i = pl.program_id(0)
    dv = dinv_ref[...]
    s_in = sinp_ref[0:1, :] + sinp_ref[1:2, :]
    s_out = soutp_ref[0:1, :] + soutp_ref[1:2, :]
    agg = dv * s_in + dv * dv * xp_ref[...]
    cc = dv * s_out + dv * dv
    m = jnp.maximum(w1_ref[...] * agg + b1_ref[...], 0.0)
    w = lax.dot_general(m, cc, (((1,), (1,)), ((), ())),
                        preferred_element_type=jnp.float32)

    @pl.when(i == 0)
    def _():
        vacc_ref[...] = jnp.zeros_like(vacc_ref)

    vacc_ref[...] += w

    @pl.when(i == nsteps - 1)
    def _():
        out_ref[...] = (
            lax.dot_general(vacc_ref[...] * (1.0 / N), w2_ref[...],
                            (((0,), (0,)), ((), ())),
                            preferred_element_type=jnp.float32)
            + b2_ref[...])


def kernel(x, edge_index, W1, b1, W2, b2):
    N = x.shape[0]
    E = edge_index.shape[1]
    hid = W1.shape[1]
    dout = W2.shape[1]

    NB = 7168
    NP = _cdiv(N, NB) * NB
    SL = NP // NS
    nsteps = NP // NB
    assert E % (NT * CHUNK) == 0, (E, NT * CHUNK)
    EPT = E // NT

    f32 = jnp.float32
    xf = jnp.concatenate([x[:, 0].astype(f32),
                          jnp.zeros((NP - N,), f32)]).reshape(1, NP)
    ei = edge_index.astype(jnp.int32)

    srcf, dstf = pl.pallas_call(
        _split_body,
        out_shape=(jax.ShapeDtypeStruct((E,), jnp.int32),
                   jax.ShapeDtypeStruct((E,), jnp.int32)),
    )(ei)

    degp = _make_hist_kernel(NP, SL, EPT)(dstf)

    u, dinv, dinv2 = pl.pallas_call(
        functools.partial(_norm_body, N),
        out_shape=(jax.ShapeDtypeStruct((NP,), f32),
                   jax.ShapeDtypeStruct((NP,), f32),
                   jax.ShapeDtypeStruct((1, NP), f32)),
    )(degp, xf)

    sinp, soutp = _make_edge_kernel(NP, SL, EPT)(srcf, dstf, u, dinv)

    out2d = pl.pallas_call(
        functools.partial(_dense_body, N, nsteps),
        grid=(nsteps,),
        in_specs=[
            pl.BlockSpec((NC, NB), lambda i: (0, i)),
            pl.BlockSpec((NC, NB), lambda i: (0, i)),
            pl.BlockSpec((1, NB), lambda i: (0, i)),
            pl.BlockSpec((1, NB), lambda i: (0, i)),
            pl.BlockSpec((hid, 1), lambda i: (0, 0)),
            pl.BlockSpec((hid, 1), lambda i: (0, 0)),
            pl.BlockSpec((hid, dout), lambda i: (0, 0)),
            pl.BlockSpec((1, dout), lambda i: (0, 0)),
        ],
        out_specs=pl.BlockSpec((1, dout), lambda i: (0, 0)),
        out_shape=jax.ShapeDtypeStruct((1, dout), f32),
        scratch_shapes=[pltpu.VMEM((hid, 1), f32)],
    )(sinp, soutp, dinv2, xf,
      W1.reshape(hid, 1).astype(f32), b1.reshape(hid, 1).astype(f32),
      W2.astype(f32), b2.reshape(1, dout).astype(f32))

    return out2d.reshape(dout)
